# Initial kernel scaffold; baseline (speedup 1.0000x reference)
import jax
import jax.numpy as jnp
from jax.experimental import pallas as pl


def kernel(x, conv_w_0, conv_b_0, conv_w_1, conv_b_1, fc_w_t, fc_b):
    raise NotImplementedError("write your pallas kernel here")


# speedup vs baseline: 12.7064x; 12.7064x over previous
import functools

import jax
import jax.numpy as jnp
from jax.experimental import pallas as pl
from jax.experimental.pallas import tpu as pltpu


def _conv3x3_kernel(x_ref, w_ref, b_ref, o_ref, *, H, W, C_in):
    taps = []
    for dh in range(3):
        for dw in range(3):
            taps.append(x_ref[0, dh:dh + H, dw:dw + W, :].reshape(H * W, C_in))
    xcol = jnp.concatenate(taps, axis=1)
    acc = jnp.dot(xcol, w_ref[...], preferred_element_type=jnp.float32)
    acc = acc + b_ref[...]
    o_ref[0] = acc.reshape(H, W, -1).astype(o_ref.dtype)


def _conv3x3_head_kernel(x_ref, w_ref, b_ref, fw_ref, fb_ref, o_ref, h_ref,
                         *, H, W, C_in):
    taps = []
    for dh in range(3):
        for dw in range(3):
            taps.append(x_ref[0, dh:dh + H, dw:dw + W, :].reshape(H * W, C_in))
    xcol = jnp.concatenate(taps, axis=1)
    acc = jnp.dot(xcol, w_ref[...], preferred_element_type=jnp.float32)
    acc = acc + b_ref[...]
    o_ref[0] = acc.reshape(H, W, -1).astype(o_ref.dtype)
    pooled = jnp.sum(jnp.maximum(acc, 0.0), axis=0, keepdims=True)
    h_ref[0] = (jnp.dot(pooled, fw_ref[...],
                        preferred_element_type=jnp.float32) + fb_ref[...])


def _repack_w(w):
    return jnp.transpose(w, (2, 3, 1, 0)).reshape(-1, w.shape[0])


def _conv_level0(xt_pad, w_col, b):
    N, Hp, Wp, C = xt_pad.shape
    H, W = Hp - 2, Wp - 2
    C_out = w_col.shape[1]
    body = functools.partial(_conv3x3_kernel, H=H, W=W, C_in=C)
    return pl.pallas_call(
        body,
        out_shape=jax.ShapeDtypeStruct((N, H, W, C_out), jnp.float32),
        grid=(N,),
        in_specs=[
            pl.BlockSpec((1, Hp, Wp, C), lambda n: (n, 0, 0, 0)),
            pl.BlockSpec((9 * C, C_out), lambda n: (0, 0)),
            pl.BlockSpec((1, C_out), lambda n: (0, 0)),
        ],
        out_specs=pl.BlockSpec((1, H, W, C_out), lambda n: (n, 0, 0, 0)),
        compiler_params=pltpu.CompilerParams(
            dimension_semantics=("parallel",)),
    )(xt_pad, w_col, b.reshape(1, C_out))


def _conv_level1_with_head(xt_pad, w_col, b, fc_w_t, fc_b):
    N, Hp, Wp, C = xt_pad.shape
    H, W = Hp - 2, Wp - 2
    C_out = w_col.shape[1]
    V = fc_w_t.shape[1]
    body = functools.partial(_conv3x3_head_kernel, H=H, W=W, C_in=C)
    return pl.pallas_call(
        body,
        out_shape=(
            jax.ShapeDtypeStruct((N, H, W, C_out), jnp.float32),
            jax.ShapeDtypeStruct((N, 1, V), jnp.float32),
        ),
        grid=(N,),
        in_specs=[
            pl.BlockSpec((1, Hp, Wp, C), lambda n: (n, 0, 0, 0)),
            pl.BlockSpec((9 * C, C_out), lambda n: (0, 0)),
            pl.BlockSpec((1, C_out), lambda n: (0, 0)),
            pl.BlockSpec((C_out, V), lambda n: (0, 0)),
            pl.BlockSpec((1, V), lambda n: (0, 0)),
        ],
        out_specs=(
            pl.BlockSpec((1, H, W, C_out), lambda n: (n, 0, 0, 0)),
            pl.BlockSpec((1, 1, V), lambda n: (n, 0, 0)),
        ),
        compiler_params=pltpu.CompilerParams(
            dimension_semantics=("parallel",)),
    )(xt_pad, w_col, b.reshape(1, C_out), fc_w_t, fc_b.reshape(1, V))


def kernel(x, conv_w_0, conv_b_0, conv_w_1, conv_b_1, fc_w_t, fc_b):
    N, C0, H, W = x.shape
    C1 = conv_w_1.shape[0]

    xt = jnp.transpose(x, (0, 2, 3, 1))
    xt_pad = jnp.pad(xt, ((0, 0), (1, 1), (1, 1), (0, 0)))
    feat0 = _conv_level0(xt_pad, _repack_w(conv_w_0), conv_b_0)

    p = xt.reshape(N, H // 2, 2, W // 2, 2, C0).mean(axis=(2, 4))
    p_pad = jnp.pad(p, ((0, 0), (1, 1), (1, 1), (0, 0)))
    w1_folded = conv_w_1[:, :C0] + conv_w_1[:, C0:]
    feat1, head = _conv_level1_with_head(
        p_pad, _repack_w(w1_folded), conv_b_1, fc_w_t, fc_b)
    head = head.reshape(N, -1)

    f0 = jnp.transpose(feat0, (0, 3, 1, 2)).astype(x.dtype)
    f1 = jnp.transpose(feat1, (0, 3, 1, 2)).astype(x.dtype)
    return [f1, f0], head


# speedup vs baseline: 18.5614x; 1.4608x over previous
import functools

import jax
import jax.numpy as jnp
from jax.experimental import pallas as pl
from jax.experimental.pallas import tpu as pltpu


def _im2col(get_tap, H, W, C):
    taps = []
    for dh in range(3):
        for dw in range(3):
            taps.append(get_tap(dh, dw).reshape(H * W, C))
    return jnp.concatenate(taps, axis=1)


def _fused_kernel(x_ref, w0_ref, b0_ref, w1_ref, b1_ref, fw_ref, fb_ref,
                  f0_ref, f1_ref, h_ref, p_scr, *, H, W, C):
    H1, W1 = H // 2, W // 2

    CB = H // 4
    for c in range(4):
        hs = c * CB
        xb = x_ref[0, hs:hs + CB + 2, :, :].astype(jnp.bfloat16)
        xcol0 = _im2col(lambda dh, dw: xb[dh:dh + CB, dw:dw + W, :], CB, W, C)
        acc0 = jnp.dot(xcol0, w0_ref[...], preferred_element_type=jnp.float32)
        f0_ref[0, hs:hs + CB] = (acc0 + b0_ref[...]).reshape(CB, W, -1)

    ev = pl.Slice(1, H1, 2)
    od = pl.Slice(2, H1, 2)
    p = (x_ref[0, ev, ev, :] + x_ref[0, ev, od, :] +
         x_ref[0, od, ev, :] + x_ref[0, od, od, :]) * 0.25
    p_scr[...] = jnp.pad(p.astype(jnp.bfloat16),
                         ((1, 1), (1, 1), (0, 0)))

    xcol1 = _im2col(lambda dh, dw: p_scr[dh:dh + H1, dw:dw + W1, :],
                    H1, W1, C)
    acc1 = jnp.dot(xcol1, w1_ref[...], preferred_element_type=jnp.float32)
    acc1 = acc1 + b1_ref[...]
    f1_ref[0] = acc1.reshape(H1, W1, -1)

    pooled = jnp.sum(jnp.maximum(acc1, 0.0), axis=0, keepdims=True)
    h_ref[0] = (jnp.dot(pooled, fw_ref[...],
                        preferred_element_type=jnp.float32) + fb_ref[...])


def _repack_w(w):
    return jnp.transpose(w, (2, 3, 1, 0)).reshape(-1, w.shape[0])


def kernel(x, conv_w_0, conv_b_0, conv_w_1, conv_b_1, fc_w_t, fc_b):
    N, C0, H, W = x.shape
    C1 = conv_w_1.shape[0]
    V = fc_w_t.shape[1]
    H1, W1 = H // 2, W // 2

    xt = jnp.transpose(x, (0, 2, 3, 1))
    xt_pad = jnp.pad(xt, ((0, 0), (1, 1), (1, 1), (0, 0)))

    w0 = _repack_w(conv_w_0).astype(jnp.bfloat16)
    w1 = _repack_w(conv_w_1[:, :C0] + conv_w_1[:, C0:]
                   ).astype(jnp.bfloat16)

    body = functools.partial(_fused_kernel, H=H, W=W, C=C0)
    feat0, feat1, head = pl.pallas_call(
        body,
        out_shape=(
            jax.ShapeDtypeStruct((N, H, W, C0), jnp.float32),
            jax.ShapeDtypeStruct((N, H1, W1, C1), jnp.float32),
            jax.ShapeDtypeStruct((N, 1, V), jnp.float32),
        ),
        grid=(N,),
        in_specs=[
            pl.BlockSpec((1, H + 2, W + 2, C0), lambda n: (n, 0, 0, 0)),
            pl.BlockSpec((9 * C0, C0), lambda n: (0, 0)),
            pl.BlockSpec((1, C0), lambda n: (0, 0)),
            pl.BlockSpec((9 * C0, C1), lambda n: (0, 0)),
            pl.BlockSpec((1, C1), lambda n: (0, 0)),
            pl.BlockSpec((C1, V), lambda n: (0, 0)),
            pl.BlockSpec((1, V), lambda n: (0, 0)),
        ],
        out_specs=(
            pl.BlockSpec((1, H, W, C0), lambda n: (n, 0, 0, 0)),
            pl.BlockSpec((1, H1, W1, C1), lambda n: (n, 0, 0, 0)),
            pl.BlockSpec((1, 1, V), lambda n: (n, 0, 0)),
        ),
        scratch_shapes=[pltpu.VMEM((H1 + 2, W1 + 2, C0), jnp.bfloat16)],
        compiler_params=pltpu.CompilerParams(
            dimension_semantics=("parallel",)),
    )(xt_pad, w0, conv_b_0.reshape(1, C0), w1, conv_b_1.reshape(1, C1),
      fc_w_t, fc_b.reshape(1, V))

    f0 = jnp.transpose(feat0, (0, 3, 1, 2))
    f1 = jnp.transpose(feat1, (0, 3, 1, 2))
    return [f1, f0], head.reshape(N, V)
